# Initial kernel scaffold; baseline (speedup 1.0000x reference)
import jax
import jax.numpy as jnp
from jax.experimental import pallas as pl


def kernel(x, hms):
    raise NotImplementedError("write your pallas kernel here")


# speedup vs baseline: 2902.4769x; 2902.4769x over previous
import functools

import jax
import jax.numpy as jnp
from jax import lax
from jax.experimental import pallas as pl
from jax.experimental.pallas import tpu as pltpu
from jax.experimental.pallas import tpu_sc as plsc

NC = 2
NS = 16
NW = NC * NS
L = 16

N_TOTAL = 4 * 4096 * 4096
PER_W = N_TOTAL // NW
CHUNK = 32768
NCHUNK = PER_W // CHUNK
UNROLL = 8

_MESH = plsc.VectorSubcoreMesh(
    core_axis_name="c", subcore_axis_name="s", num_cores=NC, num_subcores=NS
)


@functools.partial(
    pl.kernel,
    out_type=jax.ShapeDtypeStruct((NW, L), jnp.float32),
    mesh=_MESH,
    scratch_types=[
        pltpu.VMEM((CHUNK,), jnp.float32),
        pltpu.VMEM((CHUNK,), jnp.float32),
        pltpu.VMEM((256,), jnp.float32),
        pltpu.VMEM((L,), jnp.float32),
        pltpu.SemaphoreType.DMA,
        pltpu.SemaphoreType.DMA,
    ],
    compiler_params=pltpu.CompilerParams(needs_layout_passes=False),
)
def _hamming_partials(x_hbm, lut_hbm, out_hbm, buf0, buf1, lut_v, stage_v,
                      sem0, sem1):
    cid = lax.axis_index("c")
    sid = lax.axis_index("s")
    wid = sid * NC + cid
    base = wid * PER_W

    pltpu.sync_copy(lut_hbm, lut_v)

    bufs = (buf0, buf1)
    sems = (sem0, sem1)

    pltpu.async_copy(x_hbm.at[pl.ds(base, CHUNK)], buf0, sem0)
    pltpu.async_copy(x_hbm.at[pl.ds(base + CHUNK, CHUNK)], buf1, sem1)

    def pair_body(p, acc):
        for b in range(2):
            ci = p * 2 + b
            buf = bufs[b]
            sem = sems[b]
            pltpu.make_async_copy(x_hbm.at[pl.ds(base, CHUNK)], buf, sem).wait()

            def vec_body(j, a, buf=buf):
                for u in range(UNROLL):
                    off = (j * UNROLL + u) * L
                    xv = buf[pl.ds(off, L)]
                    t = xv + 128.0
                    t1 = xv + 127.0
                    uu = jnp.minimum(t, 255.0)
                    uu = jnp.maximum(uu, 0.0)
                    i = uu.astype(jnp.int32)
                    g = plsc.load_gather(lut_v, [i])
                    e = t1 - i.astype(jnp.float32)
                    a = a + g * e
                return a

            acc = lax.fori_loop(0, CHUNK // L // UNROLL, vec_body, acc)

            @pl.when(ci + 2 < NCHUNK)
            def _(buf=buf, sem=sem, ci=ci):
                pltpu.async_copy(
                    x_hbm.at[pl.ds(base + (ci + 2) * CHUNK, CHUNK)], buf, sem
                )
        return acc

    acc = lax.fori_loop(0, NCHUNK // 2, pair_body, jnp.zeros((L,), jnp.float32))
    stage_v[...] = acc
    pltpu.sync_copy(stage_v, out_hbm.at[wid])


def kernel(x, hms):
    hv = hms[255]
    lutg = hv - hms
    partials = _hamming_partials(x.reshape(-1), lutg)
    return hv * jnp.float32(N_TOTAL) + jnp.sum(partials)
